# baseline (speedup 1.0000x reference)
import functools
import math

import jax
import jax.numpy as jnp
from jax import lax
from jax.experimental import pallas as pl
from jax.experimental.pallas import tpu as pltpu
from jax.experimental.pallas import tpu_sc as plsc

N_SRC = 10000
N_DST = 10000
E = 320000
D = 128
H = 16
HD = D // H
NC = 2
NS = 16
HALF = D // NC
JH = HALF // 16
PER_TILE = E // NS
C = 32
N_CHUNK = PER_TILE // C
NPAIR = (N_CHUNK - 1) // 2
IB = 5
N_PAD = 10240
RPT = N_PAD // NS
L = 16

_HI = lax.Precision.HIGHEST

_GD = lax.GatherDimensionNumbers(
    offset_dims=(), collapsed_slice_dims=(0,), start_index_map=(0,))


def _perm(x, idx):
    return lax.gather(x, idx[:, None], _GD, (1,),
                      mode=lax.GatherScatterMode.PROMISE_IN_BOUNDS)


def _ln(x, g, b):
    m = jnp.mean(x, axis=-1, keepdims=True)
    v = jnp.mean((x - m) * (x - m), axis=-1, keepdims=True)
    return (x - m) * lax.rsqrt(v + 1e-5) * g + b


_BR = 1000


def _qxd_body(xdin_ref, Wemb_ref, bemb_ref, gd_ref, bd_ref, Wq_ref, bq_ref,
              q_ref, xd_ref):
    xd0 = jnp.dot(xdin_ref[...], Wemb_ref[...], precision=_HI) + bemb_ref[...]
    xd_ref[...] = xd0
    xd_n = _ln(xd0, gd_ref[...], bd_ref[...])
    q_ref[...] = (jnp.dot(xd_n, Wq_ref[...], precision=_HI)
                  + bq_ref[...]) * (1.0 / math.sqrt(HD))


def _qxd(x_dst, W_emb, b_emb, g_d, b_d, W_q, b_q):
    row = pl.BlockSpec((_BR, D), lambda i: (i, 0))
    wsp = pl.BlockSpec((D, D), lambda i: (0, 0))
    vsp = pl.BlockSpec((1, D), lambda i: (0, 0))
    out = jax.ShapeDtypeStruct((N_DST, D), jnp.float32)
    return pl.pallas_call(
        _qxd_body,
        grid=(N_DST // _BR,),
        in_specs=[row, wsp, vsp, vsp, vsp, wsp, vsp],
        out_specs=[row, row],
        out_shape=[out, out],
    )(x_dst, W_emb, b_emb, g_d, b_d, W_q, b_q)


def _kv_body(xs_ref, gs_ref, bs_ref, Wk_ref, bk_ref, Wv_ref, bv_ref, kv_ref):
    c = pl.program_id(0)
    xs_n = _ln(xs_ref[...], gs_ref[...], bs_ref[...])
    k = jnp.dot(xs_n, Wk_ref[...], precision=_HI) + bk_ref[...]
    v = jnp.dot(xs_n, Wv_ref[...], precision=_HI) + bv_ref[...]
    lo = jnp.concatenate([k[:, :HALF], v[:, :HALF]], axis=-1)
    hi = jnp.concatenate([k[:, HALF:], v[:, HALF:]], axis=-1)
    kv_ref[...] = jnp.where(c == 0, lo, hi)


def _kv(x_src, g_s, b_s, W_k, b_k, W_v, b_v):
    row = pl.BlockSpec((_BR, D), lambda c, i: (i, 0))
    wsp = pl.BlockSpec((D, D), lambda c, i: (0, 0))
    vsp = pl.BlockSpec((1, D), lambda c, i: (0, 0))
    nb = N_SRC // _BR
    return pl.pallas_call(
        _kv_body,
        grid=(NC, nb),
        in_specs=[row, vsp, vsp, wsp, vsp, wsp, vsp],
        out_specs=pl.BlockSpec((_BR, D), lambda c, i: (c * nb + i, 0)),
        out_shape=jax.ShapeDtypeStruct((NC * N_SRC, D), jnp.float32),
    )(x_src, g_s, b_s, W_k, b_k, W_v, b_v)


_BE = 2000


def _edge_body(ea_ref, tr_ref, Wa_ref, Wb_ref, be_ref, e_ref):
    c = pl.program_id(0)
    e = (jnp.dot(ea_ref[...], Wa_ref[...], precision=_HI)
         + jnp.dot(tr_ref[...], Wb_ref[...], precision=_HI)
         + be_ref[...])
    e_ref[0] = jnp.where(c == 0, e[:, :HALF], e[:, HALF:])


def _edge_proj(edge_attr, trainable, W_e, b_e):
    return pl.pallas_call(
        _edge_body,
        grid=(NC, E // _BE),
        in_specs=[pl.BlockSpec((_BE, 4), lambda c, i: (i, 0)),
                  pl.BlockSpec((_BE, 8), lambda c, i: (i, 0)),
                  pl.BlockSpec((4, D), lambda c, i: (0, 0)),
                  pl.BlockSpec((8, D), lambda c, i: (0, 0)),
                  pl.BlockSpec((1, D), lambda c, i: (0, 0))],
        out_specs=pl.BlockSpec((1, _BE, HALF), lambda c, i: (c, i, 0)),
        out_shape=jax.ShapeDtypeStruct((NC, E, HALF), jnp.float32),
    )(edge_attr, trainable, W_e[:4], W_e[4:], b_e.reshape(1, D))


def _sc_attn_body(kv_hbm, q_hbm, e_hbm, ei_hbm, nd_hbm,
                  src_blk, dst_blk, src_i0, src_i1, dst_i0, dst_i1,
                  kv_b0, kv_b1, q_b0, q_b1, e_b0, e_b1, wx_b0, wx_b1,
                  num_sh, skv0, skv1, sq0, sq1, se0, se1, ss0, ss1):
    cid = lax.axis_index("c")
    sid = lax.axis_index("s")

    src_i = (src_i0, src_i1)
    dst_i = (dst_i0, dst_i1)
    kv_b = (kv_b0, kv_b1)
    q_b = (q_b0, q_b1)
    e_b = (e_b0, e_b1)
    wx_b = (wx_b0, wx_b1)
    skv = (skv0, skv1)
    sq = (sq0, sq1)
    se = (se0, se1)
    ss = (ss0, ss1)

    iota = lax.iota(jnp.int32, L)
    half8 = lax.shift_right_logical(iota, 3)
    pair = lax.shift_right_logical(iota, 1)
    zero16 = jnp.zeros((L,), jnp.float32)
    col0 = cid * HALF
    row0 = sid * RPT
    base0 = sid * PER_TILE

    def zrow(r, carry):
        for j in range(D // L):
            wx_b0[r, pl.ds(L * j, L)] = zero16
            wx_b1[r, pl.ds(L * j, L)] = zero16
        return carry

    lax.fori_loop(0, C, zrow, 0)
    for t in range(RPT // C):
        pltpu.sync_copy(wx_b0, num_sh.at[pl.ds(row0 + t * C, C)])
    plsc.subcore_barrier()

    def issue(cc, b):
        @pl.when(cc % IB == 0)
        def _():
            blkbase = base0 + cc * C
            pltpu.sync_copy(ei_hbm.at[pl.ds(blkbase, IB * C)], src_blk)
            pltpu.sync_copy(ei_hbm.at[pl.ds(E + blkbase, IB * C)], dst_blk)

        off = (cc % IB) * C
        for t in range(C // L):
            sl = pl.ds(L * t, L)
            src_i[b][sl] = src_blk[pl.ds(off + L * t, L)] + cid * N_SRC
            dst_i[b][sl] = dst_blk[pl.ds(off + L * t, L)]
        pltpu.async_copy(kv_hbm.at[src_i[b]], kv_b[b], skv[b])
        pltpu.async_copy(q_hbm.at[dst_i[b]], q_b[b], sq[b])
        pltpu.async_copy(e_hbm.at[cid, pl.ds(base0 + cc * C, C)],
                         e_b[b], se[b])

    def wait_gathers(b):
        pltpu.make_async_copy(kv_hbm.at[src_i[b]], kv_b[b], skv[b]).wait()
        pltpu.make_async_copy(q_hbm.at[dst_i[b]], q_b[b], sq[b]).wait()
        pltpu.make_async_copy(e_hbm.at[cid, pl.ds(0, C)], e_b[b],
                              se[b]).wait()

    def wait_scatter(b):
        pltpu.make_async_copy(wx_b[b], num_sh.at[dst_i[b]], ss[b]).wait()

    def compute(b):
        @plsc.parallel_loop(0, C, 1, unroll=4)
        def edge(i):
            ejs = []
            logit = zero16
            for j in range(JH):
                sl = pl.ds(L * j, L)
                ej = e_b[b][i, sl]
                ejs.append(ej)
                p = q_b[b][i, pl.ds(col0 + L * j, L)] * (kv_b[b][i, sl] + ej)
                for bd in (1, 2, 4):
                    p = p + _perm(p, iota ^ bd)
                t = _perm(p, jnp.where(iota == 2 * j + 1, 8, 0))
                logit = jnp.where(pair == j, t, logit)
            ex = jnp.exp(logit)
            wx_b[b][i, pl.ds(HALF, L)] = jnp.where(half8 == 0, ex, 0.0)
            for j in range(JH):
                bj = _perm(ex, 2 * j + half8)
                wx_b[b][i, pl.ds(L * j, L)] = bj * (
                    kv_b[b][i, pl.ds(HALF + L * j, L)] + ejs[j])

    def scatter(b):
        pltpu.async_copy(wx_b[b], num_sh.at[dst_i[b]], ss[b], add=True)

    issue(0, 0)

    def pair_step(t, carry):
        cc0 = 2 * t
        wait_gathers(0)

        @pl.when(t > 0)
        def _():
            wait_scatter(1)

        issue(cc0 + 1, 1)
        compute(0)
        scatter(0)

        wait_gathers(1)
        wait_scatter(0)
        issue(cc0 + 2, 0)
        compute(1)
        scatter(1)
        return carry

    lax.fori_loop(0, NPAIR, pair_step, 0)

    wait_gathers(0)
    wait_scatter(1)
    compute(0)
    pltpu.sync_copy(wx_b0, num_sh.at[dst_i0], add=True)

    plsc.subcore_barrier()
    pltpu.sync_copy(num_sh.at[pl.ds(row0, RPT)],
                    nd_hbm.at[cid, pl.ds(row0, RPT)])


def _sc_attention(kv, q, e, ei):
    mesh = plsc.VectorSubcoreMesh(core_axis_name="c", subcore_axis_name="s")
    kern = pl.kernel(
        _sc_attn_body,
        out_type=jax.ShapeDtypeStruct((NC, N_PAD, D), jnp.float32),
        mesh=mesh,
        scratch_types=[
            pltpu.VMEM((IB * C,), jnp.int32),
            pltpu.VMEM((IB * C,), jnp.int32),
            pltpu.VMEM((C,), jnp.int32),
            pltpu.VMEM((C,), jnp.int32),
            pltpu.VMEM((C,), jnp.int32),
            pltpu.VMEM((C,), jnp.int32),
            pltpu.VMEM((C, D), jnp.float32),
            pltpu.VMEM((C, D), jnp.float32),
            pltpu.VMEM((C, D), jnp.float32),
            pltpu.VMEM((C, D), jnp.float32),
            pltpu.VMEM((C, HALF), jnp.float32),
            pltpu.VMEM((C, HALF), jnp.float32),
            pltpu.VMEM((C, D), jnp.float32),
            pltpu.VMEM((C, D), jnp.float32),
            pltpu.VMEM_SHARED((N_PAD, D), jnp.float32),
            pltpu.SemaphoreType.DMA,
            pltpu.SemaphoreType.DMA,
            pltpu.SemaphoreType.DMA,
            pltpu.SemaphoreType.DMA,
            pltpu.SemaphoreType.DMA,
            pltpu.SemaphoreType.DMA,
            pltpu.SemaphoreType.DMA,
            pltpu.SemaphoreType.DMA,
        ],
    )
    return kern(kv, q, e, ei)


_BP = 1000


def _post_body(nd_ref, xd_ref, S0_ref, S1_ref, S_ref, Wp_ref, bp_ref,
               g2_ref, b2_ref, Wm1_ref, bm1_ref, Wm2_ref, bm2_ref, out_ref):
    blk0 = nd_ref[0]
    blk1 = nd_ref[1]
    num = jnp.concatenate([blk0[:, :HALF], blk1[:, :HALF]], axis=-1)
    den = (jnp.dot(blk0, S0_ref[...], precision=_HI)
           + jnp.dot(blk1, S1_ref[...], precision=_HI))
    deninv = 1.0 / (den + 1e-16)
    attn = num * jnp.dot(deninv, S_ref[...], precision=_HI)
    out1 = jnp.dot(attn, Wp_ref[...], precision=_HI) + bp_ref[...] + xd_ref[...]
    h = _ln(out1, g2_ref[...], b2_ref[...])
    hm = jax.nn.gelu(jnp.dot(h, Wm1_ref[...], precision=_HI) + bm1_ref[...])
    out_ref[...] = out1 + jnp.dot(hm, Wm2_ref[...], precision=_HI) + bm2_ref[...]


def _post(nd, xd, W_p, b_p, g2, b2, W_m1, b_m1, W_m2, b_m2):
    S = jnp.repeat(jnp.eye(H, dtype=jnp.float32), HD, axis=1)
    eye8 = jnp.eye(8, dtype=jnp.float32)
    pad = jnp.zeros((8, 8), jnp.float32)
    blk = jnp.concatenate([jnp.zeros((HALF, H), jnp.float32),
                           jnp.concatenate([eye8, pad], axis=1),
                           jnp.zeros((D - HALF - 8, H), jnp.float32)], axis=0)
    S0 = blk
    S1 = jnp.roll(blk, 8, axis=1)
    MH = 4 * D
    return pl.pallas_call(
        _post_body,
        grid=(N_DST // _BP,),
        in_specs=[pl.BlockSpec((NC, _BP, D), lambda i: (0, i, 0)),
                  pl.BlockSpec((_BP, D), lambda i: (i, 0)),
                  pl.BlockSpec((D, H), lambda i: (0, 0)),
                  pl.BlockSpec((D, H), lambda i: (0, 0)),
                  pl.BlockSpec((H, D), lambda i: (0, 0)),
                  pl.BlockSpec((D, D), lambda i: (0, 0)),
                  pl.BlockSpec((1, D), lambda i: (0, 0)),
                  pl.BlockSpec((1, D), lambda i: (0, 0)),
                  pl.BlockSpec((1, D), lambda i: (0, 0)),
                  pl.BlockSpec((D, MH), lambda i: (0, 0)),
                  pl.BlockSpec((1, MH), lambda i: (0, 0)),
                  pl.BlockSpec((MH, D), lambda i: (0, 0)),
                  pl.BlockSpec((1, D), lambda i: (0, 0))],
        out_specs=pl.BlockSpec((_BP, D), lambda i: (i, 0)),
        out_shape=jax.ShapeDtypeStruct((N_DST, D), jnp.float32),
    )(nd, xd, S0, S1, S, W_p, b_p, g2, b2, W_m1, b_m1, W_m2, b_m2)


def kernel(x_src, x_dst, edge_index, edge_attr, trainable,
           W_emb_dst, b_emb_dst, g_ln_src, b_ln_src, g_ln_dst, b_ln_dst,
           W_q, b_q, W_k, b_k, W_v, b_v, W_e, b_e, W_p, b_p,
           g_ln2, b_ln2, W_m1, b_m1, W_m2, b_m2):
    r1 = lambda a: a.reshape(1, -1)
    ei = edge_index.astype(jnp.int32).reshape(2 * E)
    q, xd = _qxd(x_dst, W_emb_dst, r1(b_emb_dst), r1(g_ln_dst),
                 r1(b_ln_dst), W_q, r1(b_q))
    kv = _kv(x_src, r1(g_ln_src), r1(b_ln_src), W_k, r1(b_k), W_v, r1(b_v))
    e = _edge_proj(edge_attr, trainable, W_e, b_e)
    nd = _sc_attention(kv, q, e, ei)
    return _post(nd, xd, W_p, r1(b_p), r1(g_ln2), r1(b_ln2),
                 W_m1, r1(b_m1), W_m2, r1(b_m2))


# speedup vs baseline: 1.2854x; 1.2854x over previous
import functools
import math

import jax
import jax.numpy as jnp
from jax import lax
from jax.experimental import pallas as pl
from jax.experimental.pallas import tpu as pltpu
from jax.experimental.pallas import tpu_sc as plsc

N_SRC = 10000
N_DST = 10000
E = 320000
D = 128
H = 16
HD = D // H
NC = 2
NS = 16
HALF = D // NC
JH = HALF // 16
PER_TILE = E // NS
C = 32
N_CHUNK = PER_TILE // C
NPAIR = (N_CHUNK - 1) // 2
IB = 5
N_PAD = 10240
RPT = N_PAD // NS
L = 16

_HI = lax.Precision.HIGHEST

_GD = lax.GatherDimensionNumbers(
    offset_dims=(), collapsed_slice_dims=(0,), start_index_map=(0,))


def _perm(x, idx):
    return lax.gather(x, idx[:, None], _GD, (1,),
                      mode=lax.GatherScatterMode.PROMISE_IN_BOUNDS)


def _ln(x, g, b):
    m = jnp.mean(x, axis=-1, keepdims=True)
    v = jnp.mean((x - m) * (x - m), axis=-1, keepdims=True)
    return (x - m) * lax.rsqrt(v + 1e-5) * g + b


_BR = 1000


def _qxd_body(xdin_ref, Wemb_ref, bemb_ref, gd_ref, bd_ref, Wq_ref, bq_ref,
              q_ref, xd_ref):
    xd0 = jnp.dot(xdin_ref[...], Wemb_ref[...], precision=_HI) + bemb_ref[...]
    xd_ref[...] = xd0
    xd_n = _ln(xd0, gd_ref[...], bd_ref[...])
    q_ref[...] = (jnp.dot(xd_n, Wq_ref[...], precision=_HI)
                  + bq_ref[...]) * (1.0 / math.sqrt(HD))


def _qxd(x_dst, W_emb, b_emb, g_d, b_d, W_q, b_q):
    row = pl.BlockSpec((_BR, D), lambda i: (i, 0))
    wsp = pl.BlockSpec((D, D), lambda i: (0, 0))
    vsp = pl.BlockSpec((1, D), lambda i: (0, 0))
    out = jax.ShapeDtypeStruct((N_DST, D), jnp.float32)
    return pl.pallas_call(
        _qxd_body,
        grid=(N_DST // _BR,),
        in_specs=[row, wsp, vsp, vsp, vsp, wsp, vsp],
        out_specs=[row, row],
        out_shape=[out, out],
    )(x_dst, W_emb, b_emb, g_d, b_d, W_q, b_q)


def _kv_body(xs_ref, gs_ref, bs_ref, Wk_ref, bk_ref, Wv_ref, bv_ref, kv_ref):
    c = pl.program_id(0)
    xs_n = _ln(xs_ref[...], gs_ref[...], bs_ref[...])
    k = jnp.dot(xs_n, Wk_ref[...], precision=_HI) + bk_ref[...]
    v = jnp.dot(xs_n, Wv_ref[...], precision=_HI) + bv_ref[...]
    lo = jnp.concatenate([k[:, :HALF], v[:, :HALF]], axis=-1)
    hi = jnp.concatenate([k[:, HALF:], v[:, HALF:]], axis=-1)
    kv_ref[...] = jnp.where(c == 0, lo, hi)


def _kv(x_src, g_s, b_s, W_k, b_k, W_v, b_v):
    row = pl.BlockSpec((_BR, D), lambda c, i: (i, 0))
    wsp = pl.BlockSpec((D, D), lambda c, i: (0, 0))
    vsp = pl.BlockSpec((1, D), lambda c, i: (0, 0))
    nb = N_SRC // _BR
    return pl.pallas_call(
        _kv_body,
        grid=(NC, nb),
        in_specs=[row, vsp, vsp, wsp, vsp, wsp, vsp],
        out_specs=pl.BlockSpec((_BR, D), lambda c, i: (c * nb + i, 0)),
        out_shape=jax.ShapeDtypeStruct((NC * N_SRC, D), jnp.float32),
    )(x_src, g_s, b_s, W_k, b_k, W_v, b_v)


_BE = 2000


def _edge_body(ea_ref, tr_ref, Wa_ref, Wb_ref, be_ref, e_ref):
    c = pl.program_id(0)
    e = (jnp.dot(ea_ref[...], Wa_ref[...], precision=_HI)
         + jnp.dot(tr_ref[...], Wb_ref[...], precision=_HI)
         + be_ref[...])
    e_ref[0] = jnp.where(c == 0, e[:, :HALF], e[:, HALF:])


def _edge_proj(edge_attr, trainable, W_e, b_e):
    return pl.pallas_call(
        _edge_body,
        grid=(NC, E // _BE),
        in_specs=[pl.BlockSpec((_BE, 4), lambda c, i: (i, 0)),
                  pl.BlockSpec((_BE, 8), lambda c, i: (i, 0)),
                  pl.BlockSpec((4, D), lambda c, i: (0, 0)),
                  pl.BlockSpec((8, D), lambda c, i: (0, 0)),
                  pl.BlockSpec((1, D), lambda c, i: (0, 0))],
        out_specs=pl.BlockSpec((1, _BE, HALF), lambda c, i: (c, i, 0)),
        out_shape=jax.ShapeDtypeStruct((NC, E, HALF), jnp.float32),
    )(edge_attr, trainable, W_e[:4], W_e[4:], b_e.reshape(1, D))


def _sc_attn_body(kv_hbm, q_hbm, e_hbm, ei_hbm, nd_hbm,
                  src_blk, dst_blk, src_i0, src_i1, dst_i0, dst_i1,
                  kv_b0, kv_b1, q_b0, q_b1, e_b0, e_b1, wx_b0, wx_b1,
                  num_sh, skv0, skv1, sq0, sq1, se0, se1, ss0, ss1):
    cid = lax.axis_index("c")
    sid = lax.axis_index("s")

    src_i = (src_i0, src_i1)
    dst_i = (dst_i0, dst_i1)
    kv_b = (kv_b0, kv_b1)
    q_b = (q_b0, q_b1)
    e_b = (e_b0, e_b1)
    wx_b = (wx_b0, wx_b1)
    skv = (skv0, skv1)
    sq = (sq0, sq1)
    se = (se0, se1)
    ss = (ss0, ss1)

    iota = lax.iota(jnp.int32, L)
    half8 = lax.shift_right_logical(iota, 3)
    pair = lax.shift_right_logical(iota, 1)
    zero16 = jnp.zeros((L,), jnp.float32)
    col0 = cid * HALF
    row0 = sid * RPT
    base0 = sid * PER_TILE

    def zrow(r, carry):
        for j in range(D // L):
            wx_b0[r, pl.ds(L * j, L)] = zero16
            wx_b1[r, pl.ds(L * j, L)] = zero16
        return carry

    lax.fori_loop(0, C, zrow, 0)
    for t in range(RPT // C):
        pltpu.sync_copy(wx_b0, num_sh.at[pl.ds(row0 + t * C, C)])
    plsc.subcore_barrier()

    def issue(cc, b):
        @pl.when(cc % IB == 0)
        def _():
            blkbase = base0 + cc * C
            pltpu.sync_copy(ei_hbm.at[pl.ds(blkbase, IB * C)], src_blk)
            pltpu.sync_copy(ei_hbm.at[pl.ds(E + blkbase, IB * C)], dst_blk)

        off = (cc % IB) * C
        for t in range(C // L):
            sl = pl.ds(L * t, L)
            src_i[b][sl] = src_blk[pl.ds(off + L * t, L)] + cid * N_SRC
            dst_i[b][sl] = dst_blk[pl.ds(off + L * t, L)]
        pltpu.async_copy(kv_hbm.at[src_i[b]], kv_b[b], skv[b])
        pltpu.async_copy(q_hbm.at[dst_i[b]], q_b[b], sq[b])
        pltpu.async_copy(e_hbm.at[cid, pl.ds(base0 + cc * C, C)],
                         e_b[b], se[b])

    def wait_gathers(b):
        pltpu.make_async_copy(kv_hbm.at[src_i[b]], kv_b[b], skv[b]).wait()
        pltpu.make_async_copy(q_hbm.at[dst_i[b]], q_b[b], sq[b]).wait()
        pltpu.make_async_copy(e_hbm.at[cid, pl.ds(0, C)], e_b[b],
                              se[b]).wait()

    def wait_scatter(b):
        pltpu.make_async_copy(wx_b[b], num_sh.at[dst_i[b]], ss[b]).wait()

    def compute(b):
        @plsc.parallel_loop(0, C, 1, unroll=2)
        def edge(i):
            ejs = []
            logit = zero16
            for j in range(JH):
                sl = pl.ds(L * j, L)
                ej = e_b[b][i, sl]
                ejs.append(ej)
                p = q_b[b][i, pl.ds(col0 + L * j, L)] * (kv_b[b][i, sl] + ej)
                for bd in (1, 2, 4):
                    p = p + _perm(p, iota ^ bd)
                t = _perm(p, jnp.where(iota == 2 * j + 1, 8, 0))
                logit = jnp.where(pair == j, t, logit)
            ex = jnp.exp(logit)
            wx_b[b][i, pl.ds(HALF, L)] = jnp.where(half8 == 0, ex, 0.0)
            for j in range(JH):
                bj = _perm(ex, 2 * j + half8)
                wx_b[b][i, pl.ds(L * j, L)] = bj * (
                    kv_b[b][i, pl.ds(HALF + L * j, L)] + ejs[j])

    def scatter(b):
        pltpu.async_copy(wx_b[b], num_sh.at[dst_i[b]], ss[b], add=True)

    issue(0, 0)

    def pair_step(t, carry):
        cc0 = 2 * t
        wait_gathers(0)

        @pl.when(t > 0)
        def _():
            wait_scatter(1)

        issue(cc0 + 1, 1)
        compute(0)
        scatter(0)

        wait_gathers(1)
        wait_scatter(0)
        issue(cc0 + 2, 0)
        compute(1)
        scatter(1)
        return carry

    lax.fori_loop(0, NPAIR, pair_step, 0)

    wait_gathers(0)
    wait_scatter(1)
    compute(0)
    pltpu.sync_copy(wx_b0, num_sh.at[dst_i0], add=True)

    plsc.subcore_barrier()
    pltpu.sync_copy(num_sh.at[pl.ds(row0, RPT)],
                    nd_hbm.at[cid, pl.ds(row0, RPT)])


def _sc_attention(kv, q, e, ei):
    mesh = plsc.VectorSubcoreMesh(core_axis_name="c", subcore_axis_name="s")
    kern = pl.kernel(
        _sc_attn_body,
        out_type=jax.ShapeDtypeStruct((NC, N_PAD, D), jnp.float32),
        mesh=mesh,
        scratch_types=[
            pltpu.VMEM((IB * C,), jnp.int32),
            pltpu.VMEM((IB * C,), jnp.int32),
            pltpu.VMEM((C,), jnp.int32),
            pltpu.VMEM((C,), jnp.int32),
            pltpu.VMEM((C,), jnp.int32),
            pltpu.VMEM((C,), jnp.int32),
            pltpu.VMEM((C, D), jnp.float32),
            pltpu.VMEM((C, D), jnp.float32),
            pltpu.VMEM((C, D), jnp.float32),
            pltpu.VMEM((C, D), jnp.float32),
            pltpu.VMEM((C, HALF), jnp.float32),
            pltpu.VMEM((C, HALF), jnp.float32),
            pltpu.VMEM((C, D), jnp.float32),
            pltpu.VMEM((C, D), jnp.float32),
            pltpu.VMEM_SHARED((N_PAD, D), jnp.float32),
            pltpu.SemaphoreType.DMA,
            pltpu.SemaphoreType.DMA,
            pltpu.SemaphoreType.DMA,
            pltpu.SemaphoreType.DMA,
            pltpu.SemaphoreType.DMA,
            pltpu.SemaphoreType.DMA,
            pltpu.SemaphoreType.DMA,
            pltpu.SemaphoreType.DMA,
        ],
    )
    return kern(kv, q, e, ei)


_BP = 1000


def _post_body(nd_ref, xd_ref, S0_ref, S1_ref, S_ref, Wp_ref, bp_ref,
               g2_ref, b2_ref, Wm1_ref, bm1_ref, Wm2_ref, bm2_ref, out_ref):
    blk0 = nd_ref[0]
    blk1 = nd_ref[1]
    num = jnp.concatenate([blk0[:, :HALF], blk1[:, :HALF]], axis=-1)
    den = (jnp.dot(blk0, S0_ref[...], precision=_HI)
           + jnp.dot(blk1, S1_ref[...], precision=_HI))
    deninv = 1.0 / (den + 1e-16)
    attn = num * jnp.dot(deninv, S_ref[...], precision=_HI)
    out1 = jnp.dot(attn, Wp_ref[...], precision=_HI) + bp_ref[...] + xd_ref[...]
    h = _ln(out1, g2_ref[...], b2_ref[...])
    hm = jax.nn.gelu(jnp.dot(h, Wm1_ref[...], precision=_HI) + bm1_ref[...])
    out_ref[...] = out1 + jnp.dot(hm, Wm2_ref[...], precision=_HI) + bm2_ref[...]


def _post(nd, xd, W_p, b_p, g2, b2, W_m1, b_m1, W_m2, b_m2):
    S = jnp.repeat(jnp.eye(H, dtype=jnp.float32), HD, axis=1)
    eye8 = jnp.eye(8, dtype=jnp.float32)
    pad = jnp.zeros((8, 8), jnp.float32)
    blk = jnp.concatenate([jnp.zeros((HALF, H), jnp.float32),
                           jnp.concatenate([eye8, pad], axis=1),
                           jnp.zeros((D - HALF - 8, H), jnp.float32)], axis=0)
    S0 = blk
    S1 = jnp.roll(blk, 8, axis=1)
    MH = 4 * D
    return pl.pallas_call(
        _post_body,
        grid=(N_DST // _BP,),
        in_specs=[pl.BlockSpec((NC, _BP, D), lambda i: (0, i, 0)),
                  pl.BlockSpec((_BP, D), lambda i: (i, 0)),
                  pl.BlockSpec((D, H), lambda i: (0, 0)),
                  pl.BlockSpec((D, H), lambda i: (0, 0)),
                  pl.BlockSpec((H, D), lambda i: (0, 0)),
                  pl.BlockSpec((D, D), lambda i: (0, 0)),
                  pl.BlockSpec((1, D), lambda i: (0, 0)),
                  pl.BlockSpec((1, D), lambda i: (0, 0)),
                  pl.BlockSpec((1, D), lambda i: (0, 0)),
                  pl.BlockSpec((D, MH), lambda i: (0, 0)),
                  pl.BlockSpec((1, MH), lambda i: (0, 0)),
                  pl.BlockSpec((MH, D), lambda i: (0, 0)),
                  pl.BlockSpec((1, D), lambda i: (0, 0))],
        out_specs=pl.BlockSpec((_BP, D), lambda i: (i, 0)),
        out_shape=jax.ShapeDtypeStruct((N_DST, D), jnp.float32),
    )(nd, xd, S0, S1, S, W_p, b_p, g2, b2, W_m1, b_m1, W_m2, b_m2)


def kernel(x_src, x_dst, edge_index, edge_attr, trainable,
           W_emb_dst, b_emb_dst, g_ln_src, b_ln_src, g_ln_dst, b_ln_dst,
           W_q, b_q, W_k, b_k, W_v, b_v, W_e, b_e, W_p, b_p,
           g_ln2, b_ln2, W_m1, b_m1, W_m2, b_m2):
    r1 = lambda a: a.reshape(1, -1)
    ei = edge_index.astype(jnp.int32).reshape(2 * E)
    q, xd = _qxd(x_dst, W_emb_dst, r1(b_emb_dst), r1(g_ln_dst),
                 r1(b_ln_dst), W_q, r1(b_q))
    kv = _kv(x_src, r1(g_ln_src), r1(b_ln_src), W_k, r1(b_k), W_v, r1(b_v))
    e = _edge_proj(edge_attr, trainable, W_e, b_e)
    nd = _sc_attention(kv, q, e, ei)
    return _post(nd, xd, W_p, r1(b_p), r1(g_ln2), r1(b_ln2),
                 W_m1, r1(b_m1), W_m2, r1(b_m2))


# speedup vs baseline: 1.5068x; 1.1722x over previous
import functools
import math

import jax
import jax.numpy as jnp
from jax import lax
from jax.experimental import pallas as pl
from jax.experimental.pallas import tpu as pltpu
from jax.experimental.pallas import tpu_sc as plsc

N_SRC = 10000
N_DST = 10000
E = 320000
D = 128
H = 16
HD = D // H
NC = 2
NS = 16
HALF = D // NC
JH = HALF // 16
PER_TILE = E // NS
C = 32
N_CHUNK = PER_TILE // C
NPAIR = (N_CHUNK - 1) // 2
IB = 5
N_PAD = 10240
RPT = N_PAD // NS
L = 16

_HI = lax.Precision.HIGHEST

_GD = lax.GatherDimensionNumbers(
    offset_dims=(), collapsed_slice_dims=(0,), start_index_map=(0,))


def _perm(x, idx):
    return lax.gather(x, idx[:, None], _GD, (1,),
                      mode=lax.GatherScatterMode.PROMISE_IN_BOUNDS)


def _ln(x, g, b):
    m = jnp.mean(x, axis=-1, keepdims=True)
    v = jnp.mean((x - m) * (x - m), axis=-1, keepdims=True)
    return (x - m) * lax.rsqrt(v + 1e-5) * g + b


_BR = 1000


def _qxd_body(xdin_ref, Wemb_ref, bemb_ref, gd_ref, bd_ref, Wq_ref, bq_ref,
              q_ref, xd_ref):
    xd0 = jnp.dot(xdin_ref[...], Wemb_ref[...], precision=_HI) + bemb_ref[...]
    xd_ref[...] = xd0
    xd_n = _ln(xd0, gd_ref[...], bd_ref[...])
    q_ref[...] = (jnp.dot(xd_n, Wq_ref[...], precision=_HI)
                  + bq_ref[...]) * (1.0 / math.sqrt(HD))


def _qxd(x_dst, W_emb, b_emb, g_d, b_d, W_q, b_q):
    row = pl.BlockSpec((_BR, D), lambda i: (i, 0))
    wsp = pl.BlockSpec((D, D), lambda i: (0, 0))
    vsp = pl.BlockSpec((1, D), lambda i: (0, 0))
    out = jax.ShapeDtypeStruct((N_DST, D), jnp.float32)
    return pl.pallas_call(
        _qxd_body,
        grid=(N_DST // _BR,),
        in_specs=[row, wsp, vsp, vsp, vsp, wsp, vsp],
        out_specs=[row, row],
        out_shape=[out, out],
    )(x_dst, W_emb, b_emb, g_d, b_d, W_q, b_q)


def _kv_body(xs_ref, gs_ref, bs_ref, Wk_ref, bk_ref, Wv_ref, bv_ref, kv_ref):
    c = pl.program_id(0)
    xs_n = _ln(xs_ref[...], gs_ref[...], bs_ref[...])
    k = jnp.dot(xs_n, Wk_ref[...], precision=_HI) + bk_ref[...]
    v = jnp.dot(xs_n, Wv_ref[...], precision=_HI) + bv_ref[...]
    lo = jnp.concatenate([k[:, :HALF], v[:, :HALF]], axis=-1)
    hi = jnp.concatenate([k[:, HALF:], v[:, HALF:]], axis=-1)
    kv_ref[...] = jnp.where(c == 0, lo, hi)


def _kv(x_src, g_s, b_s, W_k, b_k, W_v, b_v):
    row = pl.BlockSpec((_BR, D), lambda c, i: (i, 0))
    wsp = pl.BlockSpec((D, D), lambda c, i: (0, 0))
    vsp = pl.BlockSpec((1, D), lambda c, i: (0, 0))
    nb = N_SRC // _BR
    return pl.pallas_call(
        _kv_body,
        grid=(NC, nb),
        in_specs=[row, vsp, vsp, wsp, vsp, wsp, vsp],
        out_specs=pl.BlockSpec((_BR, D), lambda c, i: (c * nb + i, 0)),
        out_shape=jax.ShapeDtypeStruct((NC * N_SRC, D), jnp.float32),
    )(x_src, g_s, b_s, W_k, b_k, W_v, b_v)


_BE = 2000


def _edge_body(ef_ref, W_ref, be_ref, e_ref):
    e_ref[0] = (jnp.dot(ef_ref[...], W_ref[0], precision=_HI) + be_ref[0])


def _edge_proj(edge_attr, trainable, W_e, b_e):
    ef = jnp.concatenate(
        [edge_attr, trainable, jnp.zeros((E, 4), jnp.float32)], axis=1)
    W16 = jnp.concatenate([W_e, jnp.zeros((4, D), jnp.float32)], axis=0)
    Ws = jnp.stack([W16[:, :HALF], W16[:, HALF:]])
    bs = jnp.stack([b_e[None, :HALF], b_e[None, HALF:]])
    return pl.pallas_call(
        _edge_body,
        grid=(NC, E // _BE),
        in_specs=[pl.BlockSpec((_BE, 16), lambda c, i: (i, 0)),
                  pl.BlockSpec((1, 16, HALF), lambda c, i: (c, 0, 0)),
                  pl.BlockSpec((1, 1, HALF), lambda c, i: (c, 0, 0))],
        out_specs=pl.BlockSpec((1, _BE, HALF), lambda c, i: (c, i, 0)),
        out_shape=jax.ShapeDtypeStruct((NC, E, HALF), jnp.float32),
    )(ef, Ws, bs)


def _sc_attn_body(kv_hbm, q_hbm, e_hbm, ei_hbm, nd_hbm,
                  src_blk, dst_blk, src_i0, src_i1, dst_i0, dst_i1,
                  kv_b0, kv_b1, q_b0, q_b1, e_b0, e_b1, wx_b0, wx_b1,
                  num_sh, skv0, skv1, sq0, sq1, se0, se1, ss0, ss1):
    cid = lax.axis_index("c")
    sid = lax.axis_index("s")

    src_i = (src_i0, src_i1)
    dst_i = (dst_i0, dst_i1)
    kv_b = (kv_b0, kv_b1)
    q_b = (q_b0, q_b1)
    e_b = (e_b0, e_b1)
    wx_b = (wx_b0, wx_b1)
    skv = (skv0, skv1)
    sq = (sq0, sq1)
    se = (se0, se1)
    ss = (ss0, ss1)

    iota = lax.iota(jnp.int32, L)
    half8 = lax.shift_right_logical(iota, 3)
    pair = lax.shift_right_logical(iota, 1)
    zero16 = jnp.zeros((L,), jnp.float32)
    col0 = cid * HALF
    row0 = sid * RPT
    base0 = sid * PER_TILE

    def zrow(r, carry):
        for j in range(D // L):
            wx_b0[r, pl.ds(L * j, L)] = zero16
            wx_b1[r, pl.ds(L * j, L)] = zero16
        return carry

    lax.fori_loop(0, C, zrow, 0)
    for t in range(RPT // C):
        pltpu.sync_copy(wx_b0, num_sh.at[pl.ds(row0 + t * C, C)])
    plsc.subcore_barrier()

    def issue(cc, b):
        @pl.when(cc % IB == 0)
        def _():
            blkbase = base0 + cc * C
            pltpu.sync_copy(ei_hbm.at[pl.ds(blkbase, IB * C)], src_blk)
            pltpu.sync_copy(ei_hbm.at[pl.ds(E + blkbase, IB * C)], dst_blk)

        off = (cc % IB) * C
        for t in range(C // L):
            sl = pl.ds(L * t, L)
            src_i[b][sl] = src_blk[pl.ds(off + L * t, L)] + cid * N_SRC
            dst_i[b][sl] = dst_blk[pl.ds(off + L * t, L)]
        pltpu.async_copy(kv_hbm.at[src_i[b]], kv_b[b], skv[b])
        pltpu.async_copy(q_hbm.at[dst_i[b]], q_b[b], sq[b])
        pltpu.async_copy(e_hbm.at[cid, pl.ds(base0 + cc * C, C)],
                         e_b[b], se[b])

    def wait_gathers(b):
        pltpu.make_async_copy(kv_hbm.at[src_i[b]], kv_b[b], skv[b]).wait()
        pltpu.make_async_copy(q_hbm.at[dst_i[b]], q_b[b], sq[b]).wait()
        pltpu.make_async_copy(e_hbm.at[cid, pl.ds(0, C)], e_b[b],
                              se[b]).wait()

    def wait_scatter(b):
        pltpu.make_async_copy(wx_b[b], num_sh.at[dst_i[b]], ss[b]).wait()

    def compute(b):
        @plsc.parallel_loop(0, C, 1, unroll=2)
        def edge(i):
            ejs = []
            logit = zero16
            for j in range(JH):
                sl = pl.ds(L * j, L)
                ej = e_b[b][i, sl]
                ejs.append(ej)
                p = q_b[b][i, pl.ds(col0 + L * j, L)] * (kv_b[b][i, sl] + ej)
                for bd in (1, 2, 4):
                    p = p + _perm(p, iota ^ bd)
                t = _perm(p, jnp.where(iota == 2 * j + 1, 8, 0))
                logit = jnp.where(pair == j, t, logit)
            ex = jnp.exp(logit)
            wx_b[b][i, pl.ds(HALF, L)] = jnp.where(half8 == 0, ex, 0.0)
            for j in range(JH):
                bj = _perm(ex, 2 * j + half8)
                wx_b[b][i, pl.ds(L * j, L)] = bj * (
                    kv_b[b][i, pl.ds(HALF + L * j, L)] + ejs[j])

    def scatter(b):
        pltpu.async_copy(wx_b[b], num_sh.at[dst_i[b]], ss[b], add=True)

    issue(0, 0)

    def pair_step(t, carry):
        cc0 = 2 * t
        wait_gathers(0)

        @pl.when(t > 0)
        def _():
            wait_scatter(1)

        issue(cc0 + 1, 1)
        compute(0)
        scatter(0)

        wait_gathers(1)
        wait_scatter(0)
        issue(cc0 + 2, 0)
        compute(1)
        scatter(1)
        return carry

    lax.fori_loop(0, NPAIR, pair_step, 0)

    wait_gathers(0)
    wait_scatter(1)
    compute(0)
    pltpu.sync_copy(wx_b0, num_sh.at[dst_i0], add=True)

    plsc.subcore_barrier()
    pltpu.sync_copy(num_sh.at[pl.ds(row0, RPT)],
                    nd_hbm.at[cid, pl.ds(row0, RPT)])


def _sc_attention(kv, q, e, ei):
    mesh = plsc.VectorSubcoreMesh(core_axis_name="c", subcore_axis_name="s")
    kern = pl.kernel(
        _sc_attn_body,
        out_type=jax.ShapeDtypeStruct((NC, N_PAD, D), jnp.float32),
        mesh=mesh,
        scratch_types=[
            pltpu.VMEM((IB * C,), jnp.int32),
            pltpu.VMEM((IB * C,), jnp.int32),
            pltpu.VMEM((C,), jnp.int32),
            pltpu.VMEM((C,), jnp.int32),
            pltpu.VMEM((C,), jnp.int32),
            pltpu.VMEM((C,), jnp.int32),
            pltpu.VMEM((C, D), jnp.float32),
            pltpu.VMEM((C, D), jnp.float32),
            pltpu.VMEM((C, D), jnp.float32),
            pltpu.VMEM((C, D), jnp.float32),
            pltpu.VMEM((C, HALF), jnp.float32),
            pltpu.VMEM((C, HALF), jnp.float32),
            pltpu.VMEM((C, D), jnp.float32),
            pltpu.VMEM((C, D), jnp.float32),
            pltpu.VMEM_SHARED((N_PAD, D), jnp.float32),
            pltpu.SemaphoreType.DMA,
            pltpu.SemaphoreType.DMA,
            pltpu.SemaphoreType.DMA,
            pltpu.SemaphoreType.DMA,
            pltpu.SemaphoreType.DMA,
            pltpu.SemaphoreType.DMA,
            pltpu.SemaphoreType.DMA,
            pltpu.SemaphoreType.DMA,
        ],
    )
    return kern(kv, q, e, ei)


_BP = 1000


def _post_body(nd_ref, xd_ref, S0_ref, S1_ref, S_ref, Wp_ref, bp_ref,
               g2_ref, b2_ref, Wm1_ref, bm1_ref, Wm2_ref, bm2_ref, out_ref):
    blk0 = nd_ref[0]
    blk1 = nd_ref[1]
    num = jnp.concatenate([blk0[:, :HALF], blk1[:, :HALF]], axis=-1)
    den = (jnp.dot(blk0, S0_ref[...], precision=_HI)
           + jnp.dot(blk1, S1_ref[...], precision=_HI))
    deninv = 1.0 / (den + 1e-16)
    attn = num * jnp.dot(deninv, S_ref[...], precision=_HI)
    out1 = jnp.dot(attn, Wp_ref[...], precision=_HI) + bp_ref[...] + xd_ref[...]
    h = _ln(out1, g2_ref[...], b2_ref[...])
    hm = jax.nn.gelu(jnp.dot(h, Wm1_ref[...], precision=_HI) + bm1_ref[...])
    out_ref[...] = out1 + jnp.dot(hm, Wm2_ref[...], precision=_HI) + bm2_ref[...]


def _post(nd, xd, W_p, b_p, g2, b2, W_m1, b_m1, W_m2, b_m2):
    S = jnp.repeat(jnp.eye(H, dtype=jnp.float32), HD, axis=1)
    eye8 = jnp.eye(8, dtype=jnp.float32)
    pad = jnp.zeros((8, 8), jnp.float32)
    blk = jnp.concatenate([jnp.zeros((HALF, H), jnp.float32),
                           jnp.concatenate([eye8, pad], axis=1),
                           jnp.zeros((D - HALF - 8, H), jnp.float32)], axis=0)
    S0 = blk
    S1 = jnp.roll(blk, 8, axis=1)
    MH = 4 * D
    return pl.pallas_call(
        _post_body,
        grid=(N_DST // _BP,),
        in_specs=[pl.BlockSpec((NC, _BP, D), lambda i: (0, i, 0)),
                  pl.BlockSpec((_BP, D), lambda i: (i, 0)),
                  pl.BlockSpec((D, H), lambda i: (0, 0)),
                  pl.BlockSpec((D, H), lambda i: (0, 0)),
                  pl.BlockSpec((H, D), lambda i: (0, 0)),
                  pl.BlockSpec((D, D), lambda i: (0, 0)),
                  pl.BlockSpec((1, D), lambda i: (0, 0)),
                  pl.BlockSpec((1, D), lambda i: (0, 0)),
                  pl.BlockSpec((1, D), lambda i: (0, 0)),
                  pl.BlockSpec((D, MH), lambda i: (0, 0)),
                  pl.BlockSpec((1, MH), lambda i: (0, 0)),
                  pl.BlockSpec((MH, D), lambda i: (0, 0)),
                  pl.BlockSpec((1, D), lambda i: (0, 0))],
        out_specs=pl.BlockSpec((_BP, D), lambda i: (i, 0)),
        out_shape=jax.ShapeDtypeStruct((N_DST, D), jnp.float32),
    )(nd, xd, S0, S1, S, W_p, b_p, g2, b2, W_m1, b_m1, W_m2, b_m2)


def kernel(x_src, x_dst, edge_index, edge_attr, trainable,
           W_emb_dst, b_emb_dst, g_ln_src, b_ln_src, g_ln_dst, b_ln_dst,
           W_q, b_q, W_k, b_k, W_v, b_v, W_e, b_e, W_p, b_p,
           g_ln2, b_ln2, W_m1, b_m1, W_m2, b_m2):
    r1 = lambda a: a.reshape(1, -1)
    ei = edge_index.astype(jnp.int32).reshape(2 * E)
    q, xd = _qxd(x_dst, W_emb_dst, r1(b_emb_dst), r1(g_ln_dst),
                 r1(b_ln_dst), W_q, r1(b_q))
    kv = _kv(x_src, r1(g_ln_src), r1(b_ln_src), W_k, r1(b_k), W_v, r1(b_v))
    e = _edge_proj(edge_attr, trainable, W_e, b_e)
    nd = _sc_attention(kv, q, e, ei)
    return _post(nd, xd, W_p, r1(b_p), r1(g_ln2), r1(b_ln2),
                 W_m1, r1(b_m1), W_m2, r1(b_m2))
